# baseline (speedup 1.0000x reference)
import functools

import jax
import jax.numpy as jnp
from jax import lax
from jax.experimental import pallas as pl
from jax.experimental.pallas import tpu as pltpu
from jax.experimental.pallas import tpu_sc as plsc

VOCAB = 100000
EMBED_DIM = 64
HIDDEN_DIM = 128
OUTPUT_DIM = 5
BATCH = 16384
HIST = 200

NUM_CORES = 2
NUM_SUBCORES = 16
NUM_WORKERS = NUM_CORES * NUM_SUBCORES
ROWS_PER_WORKER = BATCH // NUM_WORKERS
IB = 32
NUM_BLOCKS = ROWS_PER_WORKER // IB
NBUF = 4
INV_HIST = 1.0 / HIST
LANES = 16
NVEC = EMBED_DIM // LANES
PACKED_DIM = EMBED_DIM // 2


PACK_ROWS_PER_WORKER = VOCAB // NUM_WORKERS
PACK_CHUNK_ROWS = 125
PACK_CHUNKS = PACK_ROWS_PER_WORKER // PACK_CHUNK_ROWS


def _rne_bf16(u):
    return (u + jnp.uint32(0x7FFF) + ((u >> 16) & jnp.uint32(1))) >> 16


def _sc_pack_body(emb_hbm, out_hbm, in_v, out_v, sem0, sem1):
    wid = lax.axis_index("s") * NUM_CORES + lax.axis_index("c")
    row_base = wid * PACK_ROWS_PER_WORKER
    sems = (sem0, sem1)

    def fire(row0, par):
        return pltpu.async_copy(
            emb_hbm.at[pl.ds(row0 * EMBED_DIM,
                             PACK_CHUNK_ROWS * EMBED_DIM)],
            in_v.at[par], sems[par])

    def pack_chunk(row0, par):
        def row_body(r, carry):
            for j16 in range(2):
                lo = lax.bitcast_convert_type(
                    in_v[par, pl.ds(r * EMBED_DIM + j16 * LANES, LANES)],
                    jnp.uint32)
                hi = lax.bitcast_convert_type(
                    in_v[par, pl.ds(r * EMBED_DIM + PACKED_DIM
                                    + j16 * LANES, LANES)],
                    jnp.uint32)
                packed = _rne_bf16(lo) | (_rne_bf16(hi) << 16)
                out_v[par, pl.ds(r * PACKED_DIM + j16 * LANES, LANES)] = (
                    lax.bitcast_convert_type(packed, jnp.int32))
            return carry

        lax.fori_loop(0, PACK_CHUNK_ROWS, row_body, 0)
        pltpu.sync_copy(
            out_v.at[par],
            out_hbm.at[pl.ds(row0 * PACKED_DIM,
                             PACK_CHUNK_ROWS * PACKED_DIM)])

    pending = fire(row_base, 0)
    for c in range(PACK_CHUNKS):
        cur = pending
        if c + 1 < PACK_CHUNKS:
            pending = fire(row_base + (c + 1) * PACK_CHUNK_ROWS,
                           (c + 1) % 2)
        cur.wait()
        pack_chunk(row_base + c * PACK_CHUNK_ROWS, c % 2)


def _sc_pack(emb_flat):
    mesh = plsc.VectorSubcoreMesh(core_axis_name="c", subcore_axis_name="s")
    f = pl.kernel(
        _sc_pack_body,
        mesh=mesh,
        compiler_params=pltpu.CompilerParams(use_tc_tiling_on_sc=False),
        out_type=jax.ShapeDtypeStruct((VOCAB * PACKED_DIM,), jnp.int32),
        scratch_types=[
            pltpu.VMEM((2, PACK_CHUNK_ROWS * EMBED_DIM), jnp.float32),
            pltpu.VMEM((2, PACK_CHUNK_ROWS * PACKED_DIM), jnp.int32),
            pltpu.SemaphoreType.DMA,
            pltpu.SemaphoreType.DMA,
        ],
    )
    return f(emb_flat)


def _sc_mean_body(x_hbm, tab_hbm, out_hbm, idx_v, rows_v, out_v,
                  sem_i, *sems):
    wid = lax.axis_index("s") * NUM_CORES + lax.axis_index("c")
    base = wid * ROWS_PER_WORKER

    def fire(rb, par):
        c0 = pltpu.async_copy(
            tab_hbm.at[idx_v.at[rb, pl.ds(0, 128)]],
            rows_v.at[par, pl.ds(0, 128)], sems[par])
        c1 = pltpu.async_copy(
            tab_hbm.at[idx_v.at[rb, pl.ds(128, 72)]],
            rows_v.at[par, pl.ds(128, 72)], sems[par])
        return c0, c1

    def accumulate(rb, par):
        zero = jnp.zeros((LANES,), jnp.float32)
        accs0 = (zero,) * NVEC

        def t_body(i, accs):
            t0 = i * 8
            for dt in range(8):
                new = list(accs)
                for k in range(2):
                    u = rows_v[par, t0 + dt, pl.ds(k * LANES, LANES)]
                    new[k] = new[k] + lax.bitcast_convert_type(
                        u << 16, jnp.float32)
                    new[k + 2] = new[k + 2] + lax.bitcast_convert_type(
                        u & jnp.int32(-65536), jnp.float32)
                accs = tuple(new)
            return accs

        accs = lax.fori_loop(0, HIST // 8, t_body, accs0)
        for d in range(NVEC):
            out_v[rb, pl.ds(d * LANES, LANES)] = accs[d] * INV_HIST

    def blk_body(blk, carry):
        rbase = base + blk * IB
        pltpu.sync_copy(x_hbm.at[pl.ds(rbase, IB), :], idx_v)
        pending = {rb: fire(rb, rb % NBUF) for rb in range(NBUF - 1)}
        for rb in range(IB):
            nxt = rb + NBUF - 1
            if nxt < IB:
                pending[nxt] = fire(nxt, nxt % NBUF)
            c0, c1 = pending.pop(rb)
            c0.wait()
            c1.wait()
            accumulate(rb, rb % NBUF)
        pltpu.sync_copy(out_v, out_hbm.at[pl.ds(rbase, IB), :])
        return carry

    lax.fori_loop(0, NUM_BLOCKS, blk_body, 0)


def _sc_mean(x, packed_table):
    mesh = plsc.VectorSubcoreMesh(core_axis_name="c", subcore_axis_name="s")
    f = pl.kernel(
        _sc_mean_body,
        mesh=mesh,
        compiler_params=pltpu.CompilerParams(use_tc_tiling_on_sc=False),
        out_type=jax.ShapeDtypeStruct((BATCH, EMBED_DIM), jnp.float32),
        scratch_types=[
            pltpu.VMEM((IB, HIST), jnp.int32),
            pltpu.VMEM((NBUF, HIST, PACKED_DIM), jnp.int32),
            pltpu.VMEM((IB, EMBED_DIM), jnp.float32),
            pltpu.SemaphoreType.DMA,
        ] + [pltpu.SemaphoreType.DMA] * NBUF,
    )
    return f(x, packed_table)


def _mlp_body(avg_ref, w1_ref, b1_ref, w2_ref, b2_ref, out_ref):
    h = jnp.dot(avg_ref[...], w1_ref[...],
                preferred_element_type=jnp.float32) + b1_ref[...]
    h = jnp.maximum(h, 0.0)
    out_ref[...] = jnp.dot(h, w2_ref[...],
                           preferred_element_type=jnp.float32) + b2_ref[...]


def _mlp(avg, W1, b1, W2, b2):
    bs = 2048
    grid = (BATCH // bs,)
    return pl.pallas_call(
        _mlp_body,
        grid=grid,
        in_specs=[
            pl.BlockSpec((bs, EMBED_DIM), lambda i: (i, 0)),
            pl.BlockSpec((EMBED_DIM, HIDDEN_DIM), lambda i: (0, 0)),
            pl.BlockSpec((1, HIDDEN_DIM), lambda i: (0, 0)),
            pl.BlockSpec((HIDDEN_DIM, OUTPUT_DIM), lambda i: (0, 0)),
            pl.BlockSpec((1, OUTPUT_DIM), lambda i: (0, 0)),
        ],
        out_specs=pl.BlockSpec((bs, OUTPUT_DIM), lambda i: (i, 0)),
        out_shape=jax.ShapeDtypeStruct((BATCH, OUTPUT_DIM), jnp.float32),
    )(avg, W1, b1.reshape(1, HIDDEN_DIM), W2, b2.reshape(1, OUTPUT_DIM))


def kernel(x, embedding, W1, b1, W2, b2):
    packed = _sc_pack(embedding.reshape(-1)).reshape(VOCAB, PACKED_DIM)
    avg = _sc_mean(x, packed)
    return _mlp(avg, W1, b1, W2, b2)


# speedup vs baseline: 1.0702x; 1.0702x over previous
import functools

import jax
import jax.numpy as jnp
from jax import lax
from jax.experimental import pallas as pl
from jax.experimental.pallas import tpu as pltpu
from jax.experimental.pallas import tpu_sc as plsc

VOCAB = 100000
EMBED_DIM = 64
HIDDEN_DIM = 128
OUTPUT_DIM = 5
BATCH = 16384
HIST = 200

NUM_CORES = 2
NUM_SUBCORES = 16
NUM_WORKERS = NUM_CORES * NUM_SUBCORES
ROWS_PER_WORKER = BATCH // NUM_WORKERS
IB = 32
NUM_BLOCKS = ROWS_PER_WORKER // IB
NBUF = 4
INV_HIST = 1.0 / HIST
LANES = 16
NVEC = EMBED_DIM // LANES
PACKED_DIM = EMBED_DIM // 2


PACK_GROUPS = VOCAB // 8
PACK_GROUPS_BASE = PACK_GROUPS // NUM_WORKERS
PACK_EXTRA_WORKERS = PACK_GROUPS % NUM_WORKERS
PACK_CHUNK_ROWS = 120
PACK_CHUNKS = PACK_GROUPS_BASE // 15


def _rne_bf16(u):
    return (u + jnp.uint32(0x7FFF) + ((u >> 16) & jnp.uint32(1))) >> 16


def _sc_pack_body(emb_hbm, out_hbm, in_v, out_v, sem0, sem1):
    wid = lax.axis_index("s") * NUM_CORES + lax.axis_index("c")
    row_base = (wid * PACK_GROUPS_BASE
                + jnp.minimum(wid, PACK_EXTRA_WORKERS)) * 8
    sems = (sem0, sem1)

    def fire(row0, nrows, par):
        return pltpu.async_copy(
            emb_hbm.at[pl.ds(row0, nrows), :],
            in_v.at[par, pl.ds(0, nrows)], sems[par])

    def pack_chunk(row0, nrows, par):
        def row_body(r, carry):
            for j16 in range(2):
                lo = lax.bitcast_convert_type(
                    in_v[par, r, pl.ds(j16 * LANES, LANES)], jnp.uint32)
                hi = lax.bitcast_convert_type(
                    in_v[par, r, pl.ds(PACKED_DIM + j16 * LANES, LANES)],
                    jnp.uint32)
                packed = _rne_bf16(lo) | (_rne_bf16(hi) << 16)
                out_v[par, pl.ds(r * PACKED_DIM + j16 * LANES, LANES)] = (
                    lax.bitcast_convert_type(packed, jnp.int32))
            return carry

        lax.fori_loop(0, nrows, row_body, 0)
        pltpu.sync_copy(
            out_v.at[par, pl.ds(0, nrows * PACKED_DIM)],
            out_hbm.at[pl.ds(row0 * PACKED_DIM, nrows * PACKED_DIM)])

    pending = fire(row_base, PACK_CHUNK_ROWS, 0)
    for c in range(PACK_CHUNKS):
        cur = pending
        if c + 1 < PACK_CHUNKS:
            pending = fire(row_base + (c + 1) * PACK_CHUNK_ROWS,
                           PACK_CHUNK_ROWS, (c + 1) % 2)
        cur.wait()
        pack_chunk(row_base + c * PACK_CHUNK_ROWS, PACK_CHUNK_ROWS, c % 2)

    @pl.when(wid < PACK_EXTRA_WORKERS)
    def _():
        row0 = row_base + PACK_CHUNKS * PACK_CHUNK_ROWS
        fire(row0, 8, 0).wait()
        pack_chunk(row0, 8, 0)


def _sc_pack(embedding):
    mesh = plsc.VectorSubcoreMesh(core_axis_name="c", subcore_axis_name="s")
    f = pl.kernel(
        _sc_pack_body,
        mesh=mesh,
        out_type=jax.ShapeDtypeStruct((VOCAB * PACKED_DIM,), jnp.int32),
        scratch_types=[
            pltpu.VMEM((2, PACK_CHUNK_ROWS, EMBED_DIM), jnp.float32),
            pltpu.VMEM((2, PACK_CHUNK_ROWS * PACKED_DIM), jnp.int32),
            pltpu.SemaphoreType.DMA,
            pltpu.SemaphoreType.DMA,
        ],
    )
    return f(embedding)


def _sc_mean_body(x_hbm, tab_hbm, out_hbm, idx_v, rows_v, out_v,
                  sem_i, *sems):
    wid = lax.axis_index("s") * NUM_CORES + lax.axis_index("c")
    base = wid * ROWS_PER_WORKER

    def fire(rb, par):
        c0 = pltpu.async_copy(
            tab_hbm.at[idx_v.at[pl.ds(rb * HIST, 128)]],
            rows_v.at[par, pl.ds(0, 128)], sems[par])
        c1 = pltpu.async_copy(
            tab_hbm.at[idx_v.at[pl.ds(rb * HIST + 128, 72)]],
            rows_v.at[par, pl.ds(128, 72)], sems[par])
        return c0, c1

    def accumulate(rb, par):
        zero = jnp.zeros((LANES,), jnp.float32)
        accs0 = (zero,) * NVEC

        def t_body(i, accs):
            t0 = i * 8
            for dt in range(8):
                new = list(accs)
                for k in range(2):
                    u = rows_v[par, t0 + dt, pl.ds(k * LANES, LANES)]
                    new[k] = new[k] + lax.bitcast_convert_type(
                        u << 16, jnp.float32)
                    new[k + 2] = new[k + 2] + lax.bitcast_convert_type(
                        u & jnp.int32(-65536), jnp.float32)
                accs = tuple(new)
            return accs

        accs = lax.fori_loop(0, HIST // 8, t_body, accs0)
        for d in range(NVEC):
            out_v[rb, pl.ds(d * LANES, LANES)] = accs[d] * INV_HIST

    def blk_body(blk, carry):
        rbase = base + blk * IB
        pltpu.sync_copy(x_hbm.at[pl.ds(rbase * HIST, IB * HIST)], idx_v)
        pending = {rb: fire(rb, rb % NBUF) for rb in range(NBUF - 1)}
        for rb in range(IB):
            nxt = rb + NBUF - 1
            if nxt < IB:
                pending[nxt] = fire(nxt, nxt % NBUF)
            c0, c1 = pending.pop(rb)
            c0.wait()
            c1.wait()
            accumulate(rb, rb % NBUF)
        pltpu.sync_copy(out_v, out_hbm.at[pl.ds(rbase, IB), :])
        return carry

    lax.fori_loop(0, NUM_BLOCKS, blk_body, 0)


def _sc_mean(x, packed_table):
    mesh = plsc.VectorSubcoreMesh(core_axis_name="c", subcore_axis_name="s")
    f = pl.kernel(
        _sc_mean_body,
        mesh=mesh,
        compiler_params=pltpu.CompilerParams(use_tc_tiling_on_sc=False),
        out_type=jax.ShapeDtypeStruct((BATCH, EMBED_DIM), jnp.float32),
        scratch_types=[
            pltpu.VMEM((IB * HIST,), jnp.int32),
            pltpu.VMEM((NBUF, HIST, PACKED_DIM), jnp.int32),
            pltpu.VMEM((IB, EMBED_DIM), jnp.float32),
            pltpu.SemaphoreType.DMA,
        ] + [pltpu.SemaphoreType.DMA] * NBUF,
    )
    return f(x, packed_table)


def _mlp_body(avg_ref, w1_ref, b1_ref, w2_ref, b2_ref, out_ref):
    h = jnp.dot(avg_ref[...], w1_ref[...],
                preferred_element_type=jnp.float32) + b1_ref[...]
    h = jnp.maximum(h, 0.0)
    out_ref[...] = jnp.dot(h, w2_ref[...],
                           preferred_element_type=jnp.float32) + b2_ref[...]


def _mlp(avg, W1, b1, W2, b2):
    bs = 2048
    grid = (BATCH // bs,)
    return pl.pallas_call(
        _mlp_body,
        grid=grid,
        in_specs=[
            pl.BlockSpec((bs, EMBED_DIM), lambda i: (i, 0)),
            pl.BlockSpec((EMBED_DIM, HIDDEN_DIM), lambda i: (0, 0)),
            pl.BlockSpec((1, HIDDEN_DIM), lambda i: (0, 0)),
            pl.BlockSpec((HIDDEN_DIM, OUTPUT_DIM), lambda i: (0, 0)),
            pl.BlockSpec((1, OUTPUT_DIM), lambda i: (0, 0)),
        ],
        out_specs=pl.BlockSpec((bs, OUTPUT_DIM), lambda i: (i, 0)),
        out_shape=jax.ShapeDtypeStruct((BATCH, OUTPUT_DIM), jnp.float32),
    )(avg, W1, b1.reshape(1, HIDDEN_DIM), W2, b2.reshape(1, OUTPUT_DIM))


def kernel(x, embedding, W1, b1, W2, b2):
    packed = _sc_pack(embedding).reshape(VOCAB, PACKED_DIM)
    avg = _sc_mean(x.reshape(-1), packed)
    return _mlp(avg, W1, b1, W2, b2)


# speedup vs baseline: 1.1058x; 1.0332x over previous
import functools

import jax
import jax.numpy as jnp
from jax import lax
from jax.experimental import pallas as pl
from jax.experimental.pallas import tpu as pltpu
from jax.experimental.pallas import tpu_sc as plsc

VOCAB = 100000
EMBED_DIM = 64
HIDDEN_DIM = 128
OUTPUT_DIM = 5
BATCH = 16384
HIST = 200

NUM_CORES = 2
NUM_SUBCORES = 16
NUM_WORKERS = NUM_CORES * NUM_SUBCORES
ROWS_PER_WORKER = BATCH // NUM_WORKERS
IB = 32
NUM_BLOCKS = ROWS_PER_WORKER // IB
NBUF = 6
INV_HIST = 1.0 / HIST
LANES = 16
NVEC = EMBED_DIM // LANES
PACKED_DIM = EMBED_DIM // 2


PACK_GROUPS = VOCAB // 8
PACK_GROUPS_BASE = PACK_GROUPS // NUM_WORKERS
PACK_EXTRA_WORKERS = PACK_GROUPS % NUM_WORKERS
PACK_CHUNK_ROWS = 120
PACK_CHUNKS = PACK_GROUPS_BASE // 15


def _rne_bf16(u):
    return (u + jnp.uint32(0x7FFF) + ((u >> 16) & jnp.uint32(1))) >> 16


def _sc_pack_body(emb_hbm, out_hbm, in_v, out_v, sem0, sem1):
    wid = lax.axis_index("s") * NUM_CORES + lax.axis_index("c")
    row_base = (wid * PACK_GROUPS_BASE
                + jnp.minimum(wid, PACK_EXTRA_WORKERS)) * 8
    sems = (sem0, sem1)

    def fire(row0, nrows, par):
        return pltpu.async_copy(
            emb_hbm.at[pl.ds(row0, nrows), :],
            in_v.at[par, pl.ds(0, nrows)], sems[par])

    def pack_chunk(row0, nrows, par):
        def row_body(r, carry):
            for j16 in range(2):
                lo = lax.bitcast_convert_type(
                    in_v[par, r, pl.ds(j16 * LANES, LANES)], jnp.uint32)
                hi = lax.bitcast_convert_type(
                    in_v[par, r, pl.ds(PACKED_DIM + j16 * LANES, LANES)],
                    jnp.uint32)
                packed = _rne_bf16(lo) | (_rne_bf16(hi) << 16)
                out_v[par, pl.ds(r * PACKED_DIM + j16 * LANES, LANES)] = (
                    lax.bitcast_convert_type(packed, jnp.int32))
            return carry

        lax.fori_loop(0, nrows, row_body, 0)
        pltpu.sync_copy(
            out_v.at[par, pl.ds(0, nrows * PACKED_DIM)],
            out_hbm.at[pl.ds(row0 * PACKED_DIM, nrows * PACKED_DIM)])

    pending = fire(row_base, PACK_CHUNK_ROWS, 0)
    for c in range(PACK_CHUNKS):
        cur = pending
        if c + 1 < PACK_CHUNKS:
            pending = fire(row_base + (c + 1) * PACK_CHUNK_ROWS,
                           PACK_CHUNK_ROWS, (c + 1) % 2)
        cur.wait()
        pack_chunk(row_base + c * PACK_CHUNK_ROWS, PACK_CHUNK_ROWS, c % 2)

    @pl.when(wid < PACK_EXTRA_WORKERS)
    def _():
        row0 = row_base + PACK_CHUNKS * PACK_CHUNK_ROWS
        fire(row0, 8, 0).wait()
        pack_chunk(row0, 8, 0)


def _sc_pack(embedding):
    mesh = plsc.VectorSubcoreMesh(core_axis_name="c", subcore_axis_name="s")
    f = pl.kernel(
        _sc_pack_body,
        mesh=mesh,
        out_type=jax.ShapeDtypeStruct((VOCAB * PACKED_DIM,), jnp.int32),
        scratch_types=[
            pltpu.VMEM((2, PACK_CHUNK_ROWS, EMBED_DIM), jnp.float32),
            pltpu.VMEM((2, PACK_CHUNK_ROWS * PACKED_DIM), jnp.int32),
            pltpu.SemaphoreType.DMA,
            pltpu.SemaphoreType.DMA,
        ],
    )
    return f(embedding)


def _sc_mean_body(x_hbm, tab_hbm, out_hbm, idx_v, rows_v, out_v,
                  sem_i, *sems):
    wid = lax.axis_index("s") * NUM_CORES + lax.axis_index("c")
    base = wid * ROWS_PER_WORKER

    def fire(ib, rb, par):
        c0 = pltpu.async_copy(
            tab_hbm.at[idx_v.at[ib, pl.ds(rb * HIST, 128)]],
            rows_v.at[par, pl.ds(0, 128)], sems[par])
        c1 = pltpu.async_copy(
            tab_hbm.at[idx_v.at[ib, pl.ds(rb * HIST + 128, 72)]],
            rows_v.at[par, pl.ds(128, 72)], sems[par])
        return c0, c1

    def accumulate(rb, par):
        zero = jnp.zeros((LANES,), jnp.float32)
        accs0 = (zero,) * NVEC

        def t_body(i, accs):
            t0 = i * 8
            for dt in range(8):
                new = list(accs)
                for k in range(2):
                    u = rows_v[par, t0 + dt, pl.ds(k * LANES, LANES)]
                    new[k] = new[k] + lax.bitcast_convert_type(
                        u << 16, jnp.float32)
                    new[k + 2] = new[k + 2] + lax.bitcast_convert_type(
                        u & jnp.int32(-65536), jnp.float32)
                accs = tuple(new)
            return accs

        accs = lax.fori_loop(0, HIST // 8, t_body, accs0)
        for d in range(NVEC):
            out_v[rb, pl.ds(d * LANES, LANES)] = accs[d] * INV_HIST

    def next_idx_copy(blk, ib):
        rbase = base + (blk + 1) * IB
        return pltpu.make_async_copy(
            x_hbm.at[pl.ds(rbase * HIST, IB * HIST)],
            idx_v.at[ib], sem_i)

    def blk_body(blk, carry):
        rbase = base + blk * IB
        ib = lax.rem(blk, 2)

        @pl.when(blk + 1 < NUM_BLOCKS)
        def _():
            next_idx_copy(blk, 1 - ib).start()

        pending = {rb: fire(ib, rb, rb % NBUF) for rb in range(NBUF - 1)}
        for rb in range(IB):
            nxt = rb + NBUF - 1
            if nxt < IB:
                pending[nxt] = fire(ib, nxt, nxt % NBUF)
            c0, c1 = pending.pop(rb)
            c0.wait()
            c1.wait()
            accumulate(rb, rb % NBUF)
        pltpu.sync_copy(out_v, out_hbm.at[pl.ds(rbase, IB), :])

        @pl.when(blk + 1 < NUM_BLOCKS)
        def _():
            next_idx_copy(blk, 1 - ib).wait()

        return carry

    pltpu.sync_copy(x_hbm.at[pl.ds(base * HIST, IB * HIST)], idx_v.at[0])
    lax.fori_loop(0, NUM_BLOCKS, blk_body, 0)


def _sc_mean(x, packed_table):
    mesh = plsc.VectorSubcoreMesh(core_axis_name="c", subcore_axis_name="s")
    f = pl.kernel(
        _sc_mean_body,
        mesh=mesh,
        compiler_params=pltpu.CompilerParams(use_tc_tiling_on_sc=False),
        out_type=jax.ShapeDtypeStruct((BATCH, EMBED_DIM), jnp.float32),
        scratch_types=[
            pltpu.VMEM((2, IB * HIST), jnp.int32),
            pltpu.VMEM((NBUF, HIST, PACKED_DIM), jnp.int32),
            pltpu.VMEM((IB, EMBED_DIM), jnp.float32),
            pltpu.SemaphoreType.DMA,
        ] + [pltpu.SemaphoreType.DMA] * NBUF,
    )
    return f(x, packed_table)


def _mlp_body(avg_ref, w1_ref, b1_ref, w2_ref, b2_ref, out_ref):
    h = jnp.dot(avg_ref[...], w1_ref[...],
                preferred_element_type=jnp.float32) + b1_ref[...]
    h = jnp.maximum(h, 0.0)
    out_ref[...] = jnp.dot(h, w2_ref[...],
                           preferred_element_type=jnp.float32) + b2_ref[...]


def _mlp(avg, W1, b1, W2, b2):
    bs = 2048
    grid = (BATCH // bs,)
    return pl.pallas_call(
        _mlp_body,
        grid=grid,
        in_specs=[
            pl.BlockSpec((bs, EMBED_DIM), lambda i: (i, 0)),
            pl.BlockSpec((EMBED_DIM, HIDDEN_DIM), lambda i: (0, 0)),
            pl.BlockSpec((1, HIDDEN_DIM), lambda i: (0, 0)),
            pl.BlockSpec((HIDDEN_DIM, OUTPUT_DIM), lambda i: (0, 0)),
            pl.BlockSpec((1, OUTPUT_DIM), lambda i: (0, 0)),
        ],
        out_specs=pl.BlockSpec((bs, OUTPUT_DIM), lambda i: (i, 0)),
        out_shape=jax.ShapeDtypeStruct((BATCH, OUTPUT_DIM), jnp.float32),
    )(avg, W1, b1.reshape(1, HIDDEN_DIM), W2, b2.reshape(1, OUTPUT_DIM))


def kernel(x, embedding, W1, b1, W2, b2):
    packed = _sc_pack(embedding).reshape(VOCAB, PACKED_DIM)
    avg = _sc_mean(x.reshape(-1), packed)
    return _mlp(avg, W1, b1, W2, b2)


# speedup vs baseline: 1.1171x; 1.0102x over previous
import functools

import jax
import jax.numpy as jnp
from jax import lax
from jax.experimental import pallas as pl
from jax.experimental.pallas import tpu as pltpu
from jax.experimental.pallas import tpu_sc as plsc

VOCAB = 100000
EMBED_DIM = 64
HIDDEN_DIM = 128
OUTPUT_DIM = 5
BATCH = 16384
HIST = 200

NUM_CORES = 2
NUM_SUBCORES = 16
NUM_WORKERS = NUM_CORES * NUM_SUBCORES
ROWS_PER_WORKER = BATCH // NUM_WORKERS
IB = 32
NUM_BLOCKS = ROWS_PER_WORKER // IB
NBUF = 6
INV_HIST = 1.0 / HIST
LANES = 16
NVEC = EMBED_DIM // LANES
PACKED_DIM = EMBED_DIM // 2


PACK_GROUPS = VOCAB // 8
PACK_GROUPS_BASE = PACK_GROUPS // NUM_WORKERS
PACK_EXTRA_WORKERS = PACK_GROUPS % NUM_WORKERS
PACK_CHUNK_ROWS = 120
PACK_CHUNKS = PACK_GROUPS_BASE // 15


def _rne_bf16(u):
    return (u + jnp.uint32(0x7FFF) + ((u >> 16) & jnp.uint32(1))) >> 16


def _sc_pack_body(emb_hbm, out_hbm, in_v, out_v, sem0, sem1, semo0, semo1):
    wid = lax.axis_index("s") * NUM_CORES + lax.axis_index("c")
    row_base = (wid * PACK_GROUPS_BASE
                + jnp.minimum(wid, PACK_EXTRA_WORKERS)) * 8
    sems = (sem0, sem1)
    osems = (semo0, semo1)

    def fire(row0, nrows, par):
        return pltpu.async_copy(
            emb_hbm.at[pl.ds(row0, nrows), :],
            in_v.at[par, pl.ds(0, nrows)], sems[par])

    def pack_chunk(row0, nrows, par):
        def row_body(r, carry):
            for j16 in range(2):
                lo = lax.bitcast_convert_type(
                    in_v[par, r, pl.ds(j16 * LANES, LANES)], jnp.uint32)
                hi = lax.bitcast_convert_type(
                    in_v[par, r, pl.ds(PACKED_DIM + j16 * LANES, LANES)],
                    jnp.uint32)
                packed = _rne_bf16(lo) | (_rne_bf16(hi) << 16)
                out_v[par, pl.ds(r * PACKED_DIM + j16 * LANES, LANES)] = (
                    lax.bitcast_convert_type(packed, jnp.int32))
            return carry

        lax.fori_loop(0, nrows, row_body, 0)
        return pltpu.async_copy(
            out_v.at[par, pl.ds(0, nrows * PACKED_DIM)],
            out_hbm.at[pl.ds(row0 * PACKED_DIM, nrows * PACKED_DIM)],
            osems[par])

    pending = fire(row_base, PACK_CHUNK_ROWS, 0)
    out_pending = {}
    for c in range(PACK_CHUNKS):
        cur = pending
        if c + 1 < PACK_CHUNKS:
            pending = fire(row_base + (c + 1) * PACK_CHUNK_ROWS,
                           PACK_CHUNK_ROWS, (c + 1) % 2)
        cur.wait()
        if c - 2 in out_pending:
            out_pending.pop(c - 2).wait()
        out_pending[c] = pack_chunk(row_base + c * PACK_CHUNK_ROWS,
                                    PACK_CHUNK_ROWS, c % 2)
    out_pending.pop(PACK_CHUNKS - 2).wait()

    @pl.when(wid < PACK_EXTRA_WORKERS)
    def _():
        row0 = row_base + PACK_CHUNKS * PACK_CHUNK_ROWS
        fire(row0, 8, 0).wait()
        pack_chunk(row0, 8, 0).wait()

    out_pending.pop(PACK_CHUNKS - 1).wait()


def _sc_pack(embedding):
    mesh = plsc.VectorSubcoreMesh(core_axis_name="c", subcore_axis_name="s")
    f = pl.kernel(
        _sc_pack_body,
        mesh=mesh,
        out_type=jax.ShapeDtypeStruct((VOCAB * PACKED_DIM,), jnp.int32),
        scratch_types=[
            pltpu.VMEM((2, PACK_CHUNK_ROWS, EMBED_DIM), jnp.float32),
            pltpu.VMEM((2, PACK_CHUNK_ROWS * PACKED_DIM), jnp.int32),
            pltpu.SemaphoreType.DMA,
            pltpu.SemaphoreType.DMA,
            pltpu.SemaphoreType.DMA,
            pltpu.SemaphoreType.DMA,
        ],
    )
    return f(embedding)


def _sc_mean_body(x_hbm, tab_hbm, out_hbm, idx_v, rows_v, out_v,
                  sem_i, *sems):
    wid = lax.axis_index("s") * NUM_CORES + lax.axis_index("c")
    base = wid * ROWS_PER_WORKER

    def fire(ib, rb, par):
        c0 = pltpu.async_copy(
            tab_hbm.at[idx_v.at[ib, pl.ds(rb * HIST, 128)]],
            rows_v.at[par, pl.ds(0, 128)], sems[par])
        c1 = pltpu.async_copy(
            tab_hbm.at[idx_v.at[ib, pl.ds(rb * HIST + 128, 72)]],
            rows_v.at[par, pl.ds(128, 72)], sems[par])
        return c0, c1

    def accumulate(rb, par):
        zero = jnp.zeros((LANES,), jnp.float32)
        accs0 = (zero,) * NVEC

        def t_body(i, accs):
            t0 = i * 8
            for dt in range(8):
                new = list(accs)
                for k in range(2):
                    u = rows_v[par, t0 + dt, pl.ds(k * LANES, LANES)]
                    new[k] = new[k] + lax.bitcast_convert_type(
                        u << 16, jnp.float32)
                    new[k + 2] = new[k + 2] + lax.bitcast_convert_type(
                        u & jnp.int32(-65536), jnp.float32)
                accs = tuple(new)
            return accs

        accs = lax.fori_loop(0, HIST // 8, t_body, accs0)
        for d in range(NVEC):
            out_v[rb, pl.ds(d * LANES, LANES)] = accs[d] * INV_HIST

    def next_idx_copy(blk, ib):
        rbase = base + (blk + 1) * IB
        return pltpu.make_async_copy(
            x_hbm.at[pl.ds(rbase * HIST, IB * HIST)],
            idx_v.at[ib], sem_i)

    def blk_body(blk, carry):
        rbase = base + blk * IB
        ib = lax.rem(blk, 2)

        @pl.when(blk + 1 < NUM_BLOCKS)
        def _():
            next_idx_copy(blk, 1 - ib).start()

        pending = {rb: fire(ib, rb, rb % NBUF) for rb in range(NBUF - 1)}
        for rb in range(IB):
            nxt = rb + NBUF - 1
            if nxt < IB:
                pending[nxt] = fire(ib, nxt, nxt % NBUF)
            c0, c1 = pending.pop(rb)
            c0.wait()
            c1.wait()
            accumulate(rb, rb % NBUF)
        pltpu.sync_copy(out_v, out_hbm.at[pl.ds(rbase, IB), :])

        @pl.when(blk + 1 < NUM_BLOCKS)
        def _():
            next_idx_copy(blk, 1 - ib).wait()

        return carry

    pltpu.sync_copy(x_hbm.at[pl.ds(base * HIST, IB * HIST)], idx_v.at[0])
    lax.fori_loop(0, NUM_BLOCKS, blk_body, 0)


def _sc_mean(x, packed_table):
    mesh = plsc.VectorSubcoreMesh(core_axis_name="c", subcore_axis_name="s")
    f = pl.kernel(
        _sc_mean_body,
        mesh=mesh,
        compiler_params=pltpu.CompilerParams(use_tc_tiling_on_sc=False),
        out_type=jax.ShapeDtypeStruct((BATCH, EMBED_DIM), jnp.float32),
        scratch_types=[
            pltpu.VMEM((2, IB * HIST), jnp.int32),
            pltpu.VMEM((NBUF, HIST, PACKED_DIM), jnp.int32),
            pltpu.VMEM((IB, EMBED_DIM), jnp.float32),
            pltpu.SemaphoreType.DMA,
        ] + [pltpu.SemaphoreType.DMA] * NBUF,
    )
    return f(x, packed_table)


def _mlp_body(avg_ref, w1_ref, b1_ref, w2_ref, b2_ref, out_ref):
    h = jnp.dot(avg_ref[...], w1_ref[...],
                preferred_element_type=jnp.float32) + b1_ref[...]
    h = jnp.maximum(h, 0.0)
    out_ref[...] = jnp.dot(h, w2_ref[...],
                           preferred_element_type=jnp.float32) + b2_ref[...]


def _mlp(avg, W1, b1, W2, b2):
    bs = 4096
    grid = (BATCH // bs,)
    return pl.pallas_call(
        _mlp_body,
        grid=grid,
        in_specs=[
            pl.BlockSpec((bs, EMBED_DIM), lambda i: (i, 0)),
            pl.BlockSpec((EMBED_DIM, HIDDEN_DIM), lambda i: (0, 0)),
            pl.BlockSpec((1, HIDDEN_DIM), lambda i: (0, 0)),
            pl.BlockSpec((HIDDEN_DIM, OUTPUT_DIM), lambda i: (0, 0)),
            pl.BlockSpec((1, OUTPUT_DIM), lambda i: (0, 0)),
        ],
        out_specs=pl.BlockSpec((bs, OUTPUT_DIM), lambda i: (i, 0)),
        out_shape=jax.ShapeDtypeStruct((BATCH, OUTPUT_DIM), jnp.float32),
    )(avg, W1, b1.reshape(1, HIDDEN_DIM), W2, b2.reshape(1, OUTPUT_DIM))


def kernel(x, embedding, W1, b1, W2, b2):
    packed = _sc_pack(embedding).reshape(VOCAB, PACKED_DIM)
    avg = _sc_mean(x.reshape(-1), packed)
    return _mlp(avg, W1, b1, W2, b2)
